# baseline (speedup 1.0000x reference)
import functools

import jax
import jax.numpy as jnp
from jax.experimental import pallas as pl
from jax.experimental.pallas import tpu as pltpu

B, N, H = 512, 32, 100
HP = 128
G = 256


def _body(x_ref, mf_ref, me_ref, b1_ref, w2f_ref, w2e_ref, w2b_ref, cst_ref, out_ref):
    x = x_ref[...]
    xf = x.reshape(G * N, 6)
    f = jnp.dot(xf, mf_ref[...], preferred_element_type=jnp.float32) + b1_ref[...]
    e = jnp.dot(xf, me_ref[...], preferred_element_type=jnp.float32)
    bf16 = jnp.bfloat16
    fb16 = f.astype(bf16)
    eb16 = e.astype(bf16)
    s_rows = []
    for g in range(G):
        fgb = fb16[g * N:(g + 1) * N, :]
        egb = eb16[g * N:(g + 1) * N, :]
        t = [jnp.abs(fgb + egb[j:j + 1, :]) for j in range(4)]
        for j in range(4, N, 4):
            for r in range(4):
                t[r] = t[r] + jnp.abs(fgb + egb[j + r:j + r + 1, :])
        diag = fgb + egb
        sg = (bf16(0.45) * ((t[0] + t[1]) + (t[2] + t[3]))
              - (bf16(0.55) * diag + bf16(0.45) * jnp.abs(diag)))
        s_rows.append(sg)
    s = jnp.concatenate(s_rows, axis=0)
    pf = jnp.dot(f, w2f_ref[...], preferred_element_type=jnp.float32)
    pe = jnp.dot(e, w2e_ref[...], preferred_element_type=jnp.float32)
    se = (jnp.sum(pe.reshape(G, N, 6), axis=1, keepdims=True)
          + cst_ref[...])
    pt = jnp.dot(s, w2b_ref[...], preferred_element_type=jnp.float32)
    p = ((pt + pf)
         + jnp.broadcast_to(se, (G, N, 6)).reshape(G * N, 6))
    pt_ = jnp.transpose(p)
    xft = jnp.transpose(xf)
    sp = 0.1 * (jnp.maximum(pt_, 0.0) + jnp.log1p(jnp.exp(-jnp.abs(pt_))))
    upd = xft + 0.1 * pt_
    chan = jax.lax.broadcasted_iota(jnp.int32, (6, G * N), 0)
    out_t = jnp.where(chan < 4, upd, sp)
    out_ref[...] = jnp.transpose(out_t).reshape(G, N, 6)


@jax.jit
def kernel(inp, W1, b1, gamma, beta, running_mean, running_var, W2, b2):
    f32 = jnp.float32
    inp = inp.astype(f32)
    s = gamma * jax.lax.rsqrt(running_var + 1e-5)
    t = beta - s * running_mean
    w2p = (W2 * s[None, :]).astype(f32)
    cst = (N - 1.0) * (W2 @ t + b2)
    wdy = W1[:, 8]
    wdx = W1[:, 9]
    mf = jnp.concatenate([wdy[:, None], wdx[:, None], W1[:, 0:4]], axis=1)
    me = jnp.concatenate([-wdy[:, None], -wdx[:, None], W1[:, 4:8]], axis=1)
    mf_p = jnp.zeros((6, HP), f32).at[:, :H].set(mf.T)
    me_p = jnp.zeros((6, HP), f32).at[:, :H].set(me.T)
    b1_p = jnp.zeros((1, HP), f32).at[:, :H].set(b1)
    w2_p = jnp.zeros((HP, 6), f32).at[:H, :].set(w2p.T)
    w2f_p = 0.55 * N * w2_p
    w2e_p = 0.55 * w2_p
    w2b_p = w2_p.astype(jnp.bfloat16)
    cst_p = cst.reshape(1, 1, 6).astype(f32)

    out = pl.pallas_call(
        _body,
        grid=(B // G,),
        in_specs=[
            pl.BlockSpec((G, N, 6), lambda g: (g, 0, 0)),
            pl.BlockSpec((6, HP), lambda g: (0, 0)),
            pl.BlockSpec((6, HP), lambda g: (0, 0)),
            pl.BlockSpec((1, HP), lambda g: (0, 0)),
            pl.BlockSpec((HP, 6), lambda g: (0, 0)),
            pl.BlockSpec((HP, 6), lambda g: (0, 0)),
            pl.BlockSpec((HP, 6), lambda g: (0, 0)),
            pl.BlockSpec((1, 1, 6), lambda g: (0, 0, 0)),
        ],
        out_specs=pl.BlockSpec((G, N, 6), lambda g: (g, 0, 0)),
        out_shape=jax.ShapeDtypeStruct((B, N, 6), f32),
        compiler_params=pltpu.CompilerParams(
            dimension_semantics=("parallel",)),
    )(inp, mf_p, me_p, b1_p, w2f_p, w2e_p, w2b_p, cst_p)
    return out


# speedup vs baseline: 1.0060x; 1.0060x over previous
import jax
import jax.numpy as jnp
from jax.experimental import pallas as pl
from jax.experimental.pallas import tpu as pltpu

B, N, H = 512, 32, 100
HP = 128
G = 128


def _body(x_ref, mf_ref, me_ref, b1_ref, w2f_ref, w2e_ref, w2b_ref, cst_ref, out_ref):
    x = x_ref[...]
    xf = x.reshape(G * N, 6)
    f = jnp.dot(xf, mf_ref[...], preferred_element_type=jnp.float32) + b1_ref[...]
    e = jnp.dot(xf, me_ref[...], preferred_element_type=jnp.float32)
    bf16 = jnp.bfloat16
    fb16 = f.astype(bf16)
    eb16 = e.astype(bf16)
    s_rows = []
    for g in range(G):
        fgb = fb16[g * N:(g + 1) * N, :]
        egb = eb16[g * N:(g + 1) * N, :]
        t = [jnp.abs(fgb + egb[j:j + 1, :]) for j in range(4)]
        for j in range(4, N, 4):
            for r in range(4):
                t[r] = t[r] + jnp.abs(fgb + egb[j + r:j + r + 1, :])
        diag = fgb + egb
        sg = (bf16(0.45) * ((t[0] + t[1]) + (t[2] + t[3]))
              - (bf16(0.55) * diag + bf16(0.45) * jnp.abs(diag)))
        s_rows.append(sg)
    s = jnp.concatenate(s_rows, axis=0)
    pf = jnp.dot(f, w2f_ref[...], preferred_element_type=jnp.float32)
    pe = jnp.dot(e, w2e_ref[...], preferred_element_type=jnp.float32)
    se = (jnp.sum(pe.reshape(G, N, 6), axis=1, keepdims=True)
          + cst_ref[...])
    pt = jnp.dot(s, w2b_ref[...], preferred_element_type=jnp.float32)
    p = ((pt + pf)
         + jnp.broadcast_to(se, (G, N, 6)).reshape(G * N, 6))
    pt_ = jnp.transpose(p)
    xft = jnp.transpose(xf)
    sp = 0.1 * (jnp.maximum(pt_, 0.0) + jnp.log1p(jnp.exp(-jnp.abs(pt_))))
    upd = xft + 0.1 * pt_
    chan = jax.lax.broadcasted_iota(jnp.int32, (6, G * N), 0)
    out_t = jnp.where(chan < 4, upd, sp)
    out_ref[...] = jnp.transpose(out_t).reshape(G, N, 6)


@jax.jit
def kernel(inp, W1, b1, gamma, beta, running_mean, running_var, W2, b2):
    f32 = jnp.float32
    inp = inp.astype(f32)
    s = gamma * jax.lax.rsqrt(running_var + 1e-5)
    t = beta - s * running_mean
    w2p = (W2 * s[None, :]).astype(f32)
    cst = (N - 1.0) * (W2 @ t + b2)
    wdy = W1[:, 8]
    wdx = W1[:, 9]
    mf = jnp.concatenate([wdy[:, None], wdx[:, None], W1[:, 0:4]], axis=1)
    me = jnp.concatenate([-wdy[:, None], -wdx[:, None], W1[:, 4:8]], axis=1)
    mf_p = jnp.zeros((6, HP), f32).at[:, :H].set(mf.T)
    me_p = jnp.zeros((6, HP), f32).at[:, :H].set(me.T)
    b1_p = jnp.zeros((1, HP), f32).at[:, :H].set(b1)
    w2_p = jnp.zeros((HP, 6), f32).at[:H, :].set(w2p.T)
    w2f_p = 0.55 * N * w2_p
    w2e_p = 0.55 * w2_p
    w2b_p = w2_p.astype(jnp.bfloat16)
    cst_p = cst.reshape(1, 1, 6).astype(f32)

    out = pl.pallas_call(
        _body,
        grid=(B // G,),
        in_specs=[
            pl.BlockSpec((G, N, 6), lambda g: (g, 0, 0)),
            pl.BlockSpec((6, HP), lambda g: (0, 0)),
            pl.BlockSpec((6, HP), lambda g: (0, 0)),
            pl.BlockSpec((1, HP), lambda g: (0, 0)),
            pl.BlockSpec((HP, 6), lambda g: (0, 0)),
            pl.BlockSpec((HP, 6), lambda g: (0, 0)),
            pl.BlockSpec((HP, 6), lambda g: (0, 0)),
            pl.BlockSpec((1, 1, 6), lambda g: (0, 0, 0)),
        ],
        out_specs=pl.BlockSpec((G, N, 6), lambda g: (g, 0, 0)),
        out_shape=jax.ShapeDtypeStruct((B, N, 6), f32),
        compiler_params=pltpu.CompilerParams(
            dimension_semantics=("parallel",)),
    )(inp, mf_p, me_p, b1_p, w2f_p, w2e_p, w2b_p, cst_p)
    return out
